# baseline (speedup 1.0000x reference)
import jax
import jax.numpy as jnp
from jax import lax
from jax.experimental import pallas as pl
from jax.experimental.pallas import tpu as pltpu
from jax.experimental.pallas import tpu_sc as plsc

B = 16384
F = 26
K = 16
VOCAB = 100000
V = VOCAB * F
D_F = 13

NC, NS = 2, 16
NW = NC * NS
BPW = B // NW
IPW = BPW * F

GK = 8
NCHUNK = K // GK

CH = 32768
NCH = -(-V // CH)


def _make_planes_call(j, with_lin):
    if with_lin:
        def body(e2t_ref, lt_ref, *o_refs):
            for k in range(GK):
                o_refs[k][...] = e2t_ref[k, :]
            o_refs[GK][...] = lt_ref[0, :]
        n_out = GK + 1
        in_specs = [
            pl.BlockSpec((GK, CH), lambda i: (j, i)),
            pl.BlockSpec((1, CH), lambda i: (0, i)),
        ]
    else:
        def body(e2t_ref, *o_refs):
            for k in range(GK):
                o_refs[k][...] = e2t_ref[k, :]
        n_out = GK
        in_specs = [pl.BlockSpec((GK, CH), lambda i: (j, i))]
    return pl.pallas_call(
        body,
        out_shape=tuple(
            jax.ShapeDtypeStruct((V,), jnp.float32) for _ in range(n_out)),
        grid=(NCH,),
        in_specs=in_specs,
        out_specs=tuple(
            pl.BlockSpec((CH,), lambda i: (i,)) for _ in range(n_out)),
        name=f"planes{j}",
    )


_planes_calls = [_make_planes_call(j, j == NCHUNK - 1) for j in range(NCHUNK)]

_sc_params = pltpu.CompilerParams(
    needs_layout_passes=False, use_tc_tiling_on_sc=False)


def _make_sc_call(with_lin):
    npl = GK + 1 if with_lin else GK

    def body(xflat_hbm, *refs):
        planes = refs[:npl]
        if with_lin:
            acc_hbm, sq_hbm, lin_hbm = refs[npl:npl + 3]
            scratch = refs[npl + 3:]
            idx_v, val0, val1, acc_v, sq_v, lin_v, sem0, sem1 = scratch
        else:
            acc_hbm, sq_hbm = refs[npl:npl + 2]
            scratch = refs[npl + 2:]
            idx_v, val0, val1, acc_v, sq_v, sem0, sem1 = scratch

        wid = lax.axis_index("s") * NC + lax.axis_index("c")
        base = wid * BPW
        ibase = wid * IPW

        pltpu.sync_copy(xflat_hbm.at[pl.ds(ibase, IPW)], idx_v)

        iota = lax.iota(jnp.int32, 16)

        @pl.loop(0, IPW // 16)
        def _(r):
            s = pl.multiple_of(r * 16, 16)
            pos = iota + s
            off = lax.rem(pos, F) * VOCAB
            idx_v[pl.ds(s, 16)] = idx_v[pl.ds(s, 16)] + off

        bufs = (val0, val1)
        sems = (sem0, sem1)

        def gather(p):
            i = p % 2
            return pltpu.make_async_copy(planes[p].at[idx_v], bufs[i],
                                         sems[i])

        def reduce_plane(p):
            vals = bufs[p % 2]

            @pl.loop(0, BPW // 16)
            def _(t):
                b16 = iota * F + t * (16 * F)
                s = plsc.load_gather(vals, [b16])
                q = s * s
                for f in range(1, F):
                    v = plsc.load_gather(vals, [b16 + f])
                    s = s + v
                    q = q + v * v
                col = pl.multiple_of(t * 16, 16)
                if p < GK:
                    acc_v[p, pl.ds(col, 16)] = s
                    sq_v[p, pl.ds(col, 16)] = q
                else:
                    lin_v[pl.ds(col, 16)] = s

        gather(0).start()
        for p in range(npl):
            if p + 1 < npl:
                gather(p + 1).start()
            gather(p).wait()
            reduce_plane(p)

        pltpu.sync_copy(acc_v, acc_hbm.at[:, pl.ds(base, BPW)])
        pltpu.sync_copy(sq_v, sq_hbm.at[:, pl.ds(base, BPW)])
        if with_lin:
            pltpu.sync_copy(lin_v, lin_hbm.at[0, pl.ds(base, BPW)])

    out_type = [
        jax.ShapeDtypeStruct((GK, B), jnp.float32),
        jax.ShapeDtypeStruct((GK, B), jnp.float32),
    ]
    scratch = [
        pltpu.VMEM((IPW,), jnp.int32),
        pltpu.VMEM((IPW,), jnp.float32),
        pltpu.VMEM((IPW,), jnp.float32),
        pltpu.VMEM((GK, BPW), jnp.float32),
        pltpu.VMEM((GK, BPW), jnp.float32),
    ]
    if with_lin:
        out_type.append(jax.ShapeDtypeStruct((1, B), jnp.float32))
        scratch.append(pltpu.VMEM((BPW,), jnp.float32))
    scratch += [pltpu.SemaphoreType.DMA, pltpu.SemaphoreType.DMA]

    return pl.kernel(
        body,
        compiler_params=_sc_params,
        out_type=tuple(out_type),
        mesh=plsc.VectorSubcoreMesh(core_axis_name="c", subcore_axis_name="s"),
        scratch_types=scratch,
    )


_sc_call = _make_sc_call(False)
_sc_call_lin = _make_sc_call(True)

BLK = 2048


def _combine_body(*refs):
    acc_refs = refs[0:NCHUNK]
    sq_refs = refs[NCHUNK:2 * NCHUNK]
    lin_ref, dxt_ref, w2t_ref, wl_ref, b_ref, o_ref = refs[2 * NCHUNK:]
    dxt = dxt_ref[...]
    accT = jnp.concatenate([r[...] for r in acc_refs], axis=0)
    w2t = w2t_ref[...]
    wl = wl_ref[...]
    dvecT = jnp.zeros_like(accT)
    dlinT = jnp.zeros((1, BLK), jnp.float32)
    dsqT = jnp.zeros((1, BLK), jnp.float32)
    for f in range(D_F):
        row = dxt[f:f + 1, :]
        col = w2t[:, f:f + 1]
        dvecT = dvecT + col * row
        dlinT = dlinT + wl[0:1, f:f + 1] * row
        s2f = jnp.sum(col * col, axis=0, keepdims=True)
        dsqT = dsqT + s2f * (row * row)
    totT = accT + dvecT
    aT = jnp.sum(totT * totT, axis=0, keepdims=True)
    bT = sum(jnp.sum(r[...], axis=0, keepdims=True) for r in sq_refs) + dsqT
    o_ref[...] = b_ref[...] + lin_ref[...] + dlinT + 0.5 * (aT - bT)


_combine = pl.pallas_call(
    _combine_body,
    out_shape=jax.ShapeDtypeStruct((1, B), jnp.float32),
    grid=(B // BLK,),
    in_specs=(
        [pl.BlockSpec((GK, BLK), lambda i: (0, i)) for _ in range(2 * NCHUNK)]
        + [
            pl.BlockSpec((1, BLK), lambda i: (0, i)),
            pl.BlockSpec((D_F, BLK), lambda i: (0, i)),
            pl.BlockSpec((K, D_F), lambda i: (0, 0)),
            pl.BlockSpec((1, D_F), lambda i: (0, 0)),
            pl.BlockSpec((1, 1), lambda i: (0, 0)),
        ]
    ),
    out_specs=pl.BlockSpec((1, BLK), lambda i: (0, i)),
)


def kernel(sparse_x, dense_x, bias, emb_linear, dense_linear_w, emb_second,
           dense_second_w):
    x_flat = sparse_x.reshape(B * F)
    e2t = emb_second.T
    lt = emb_linear.T
    accs, sqs = [], []
    lin = None
    for j in range(NCHUNK):
        if j < NCHUNK - 1:
            planes = _planes_calls[j](e2t)
            a, q = _sc_call(x_flat, *planes)
        else:
            planes = _planes_calls[j](e2t, lt)
            a, q, lin = _sc_call_lin(x_flat, *planes)
        accs.append(a)
        sqs.append(q)
    out = _combine(
        *accs, *sqs, lin, dense_x.T,
        dense_second_w.reshape(D_F, K).T,
        dense_linear_w.reshape(1, D_F),
        bias.reshape(1, 1),
    )
    return out.reshape(B)


# speedup vs baseline: 1.0109x; 1.0109x over previous
import jax
import jax.numpy as jnp
from jax import lax
from jax.experimental import pallas as pl
from jax.experimental.pallas import tpu as pltpu
from jax.experimental.pallas import tpu_sc as plsc

B = 16384
F = 26
K = 16
VOCAB = 100000
V = VOCAB * F
D_F = 13

NC, NS = 2, 16
NW = NC * NS
BPW = B // NW
IPW = BPW * F

CH = 32768
NCH = -(-V // CH)


def _planes_body(e2t_ref, lt_ref, *o_refs):
    for k in range(K):
        o_refs[k][...] = e2t_ref[k, :]
    o_refs[K][...] = lt_ref[0, :]


_format_planes = pl.pallas_call(
    _planes_body,
    out_shape=tuple(
        jax.ShapeDtypeStruct((V,), jnp.float32) for _ in range(K + 1)),
    grid=(NCH,),
    in_specs=[
        pl.BlockSpec((K, CH), lambda i: (0, i)),
        pl.BlockSpec((1, CH), lambda i: (0, i)),
    ],
    out_specs=tuple(pl.BlockSpec((CH,), lambda i: (i,)) for _ in range(K + 1)),
)

_sc_params = pltpu.CompilerParams(
    needs_layout_passes=False, use_tc_tiling_on_sc=False)

NPL = K + 1


def _sc_body(xflat_hbm, *refs):
    planes = refs[:NPL]
    acc_hbm, sq_hbm, lin_hbm = refs[NPL:NPL + 3]
    idx_v, val0, val1, acc_v, sq_v, lin_v = refs[NPL + 3:NPL + 9]
    sem0, sem1 = refs[NPL + 9:NPL + 11]

    wid = lax.axis_index("s") * NC + lax.axis_index("c")
    base = wid * BPW
    ibase = wid * IPW

    pltpu.sync_copy(xflat_hbm.at[pl.ds(ibase, IPW)], idx_v)

    iota = lax.iota(jnp.int32, 16)

    @pl.loop(0, IPW // 16)
    def _(r):
        s = pl.multiple_of(r * 16, 16)
        pos = iota + s
        off = lax.rem(pos, F) * VOCAB
        idx_v[pl.ds(s, 16)] = idx_v[pl.ds(s, 16)] + off

    bufs = (val0, val1)
    sems = (sem0, sem1)

    def gather(p):
        i = p % 2
        return pltpu.make_async_copy(planes[p].at[idx_v], bufs[i], sems[i])

    def reduce_plane(p):
        vals = bufs[p % 2]

        @pl.loop(0, BPW // 16)
        def _(t):
            b16 = iota * F + t * (16 * F)
            s = plsc.load_gather(vals, [b16])
            q = s * s
            for f in range(1, F):
                v = plsc.load_gather(vals, [b16 + f])
                s = s + v
                q = q + v * v
            col = pl.multiple_of(t * 16, 16)
            if p < K:
                acc_v[p, pl.ds(col, 16)] = s
                sq_v[p, pl.ds(col, 16)] = q
            else:
                lin_v[pl.ds(col, 16)] = s

    gather(0).start()
    for p in range(NPL):
        if p + 1 < NPL:
            gather(p + 1).start()
        gather(p).wait()
        reduce_plane(p)

    pltpu.sync_copy(acc_v, acc_hbm.at[:, pl.ds(base, BPW)])
    pltpu.sync_copy(sq_v, sq_hbm.at[:, pl.ds(base, BPW)])
    pltpu.sync_copy(lin_v, lin_hbm.at[0, pl.ds(base, BPW)])


_sc_gather_reduce = pl.kernel(
    _sc_body,
    compiler_params=_sc_params,
    out_type=(
        jax.ShapeDtypeStruct((K, B), jnp.float32),
        jax.ShapeDtypeStruct((K, B), jnp.float32),
        jax.ShapeDtypeStruct((1, B), jnp.float32),
    ),
    mesh=plsc.VectorSubcoreMesh(core_axis_name="c", subcore_axis_name="s"),
    scratch_types=[
        pltpu.VMEM((IPW,), jnp.int32),
        pltpu.VMEM((IPW,), jnp.float32),
        pltpu.VMEM((IPW,), jnp.float32),
        pltpu.VMEM((K, BPW), jnp.float32),
        pltpu.VMEM((K, BPW), jnp.float32),
        pltpu.VMEM((BPW,), jnp.float32),
        pltpu.SemaphoreType.DMA,
        pltpu.SemaphoreType.DMA,
    ],
)

BLK = 2048


def _combine_body(acc_ref, sq_ref, lin_ref, dxt_ref, w2t_ref, wl_ref, b_ref,
                  o_ref):
    dxt = dxt_ref[...]
    accT = acc_ref[...]
    w2t = w2t_ref[...]
    wl = wl_ref[...]
    dvecT = jnp.zeros_like(accT)
    dlinT = jnp.zeros((1, BLK), jnp.float32)
    dsqT = jnp.zeros((1, BLK), jnp.float32)
    for f in range(D_F):
        row = dxt[f:f + 1, :]
        col = w2t[:, f:f + 1]
        dvecT = dvecT + col * row
        dlinT = dlinT + wl[0:1, f:f + 1] * row
        s2f = jnp.sum(col * col, axis=0, keepdims=True)
        dsqT = dsqT + s2f * (row * row)
    totT = accT + dvecT
    aT = jnp.sum(totT * totT, axis=0, keepdims=True)
    bT = jnp.sum(sq_ref[...], axis=0, keepdims=True) + dsqT
    o_ref[...] = b_ref[...] + lin_ref[...] + dlinT + 0.5 * (aT - bT)


_combine = pl.pallas_call(
    _combine_body,
    out_shape=jax.ShapeDtypeStruct((1, B), jnp.float32),
    grid=(B // BLK,),
    in_specs=[
        pl.BlockSpec((K, BLK), lambda i: (0, i)),
        pl.BlockSpec((K, BLK), lambda i: (0, i)),
        pl.BlockSpec((1, BLK), lambda i: (0, i)),
        pl.BlockSpec((D_F, BLK), lambda i: (0, i)),
        pl.BlockSpec((K, D_F), lambda i: (0, 0)),
        pl.BlockSpec((1, D_F), lambda i: (0, 0)),
        pl.BlockSpec((1, 1), lambda i: (0, 0)),
    ],
    out_specs=pl.BlockSpec((1, BLK), lambda i: (0, i)),
)


def kernel(sparse_x, dense_x, bias, emb_linear, dense_linear_w, emb_second,
           dense_second_w):
    x_flat = sparse_x.reshape(B * F)
    planes = _format_planes(emb_second.T, emb_linear.T)
    accT, sqT, lin = _sc_gather_reduce(x_flat, *planes)
    out = _combine(
        accT, sqT, lin, dense_x.T,
        dense_second_w.reshape(D_F, K).T,
        dense_linear_w.reshape(1, D_F),
        bias.reshape(1, 1),
    )
    return out.reshape(B)
